# baseline (speedup 1.0000x reference)
import functools

import jax
import jax.numpy as jnp
from jax import lax
from jax.experimental import pallas as pl
from jax.experimental.pallas import tpu as pltpu
from jax.experimental.pallas import tpu_sc as plsc

VOCAB = 1000000
EMB = 64
B = 4096
NC = 2
NS = 16
L = 16
NW = NC * NS
B_TC = 2048
B_SC = B - B_TC
BPW = B_SC // NW
NBUF = 4

_mesh = plsc.VectorSubcoreMesh(core_axis_name="c", subcore_axis_name="s")

_GATHER_DNUMS = lax.GatherDimensionNumbers(
    offset_dims=(), collapsed_slice_dims=(0,), start_index_map=(0,))


def _lane_shuffle(x, idx):
    return lax.gather(
        x, idx[:, None], _GATHER_DNUMS, slice_sizes=(1,),
        mode=lax.GatherScatterMode.PROMISE_IN_BOUNDS)


@functools.partial(
    pl.kernel,
    mesh=_mesh,
    compiler_params=pltpu.CompilerParams(needs_layout_passes=False),
    out_type=jax.ShapeDtypeStruct((B_SC,), jnp.float32),
    scratch_types=[
        pltpu.VMEM((BPW,), jnp.int32),
        pltpu.VMEM((BPW,), jnp.int32),
        pltpu.VMEM((NBUF, EMB, 128), jnp.float32),
        pltpu.VMEM((NBUF, EMB, 128), jnp.float32),
        pltpu.VMEM((BPW,), jnp.float32),
        pltpu.SemaphoreType.DMA,
        pltpu.SemaphoreType.DMA,
        pltpu.SemaphoreType.DMA,
        pltpu.SemaphoreType.DMA,
    ],
)
def _w2v_sc(tT_hbm, widx_hbm, cidx_hbm, out_hbm,
            widx_v, cidx_v, wslab_v, cslab_v, out_v, sem0, sem1, sem2, sem3):
    wid = lax.axis_index("s") * NC + lax.axis_index("c")
    base = wid * BPW
    pltpu.sync_copy(widx_hbm.at[pl.ds(base, BPW)], widx_v)
    pltpu.sync_copy(cidx_hbm.at[pl.ds(base, BPW)], cidx_v)

    sems = (sem0, sem1, sem2, sem3)
    lane = lax.iota(jnp.int32, L)
    perms = [jnp.bitwise_xor(lane, s) for s in (8, 4, 2, 1)]

    def slab_start(c):
        return pl.multiple_of((c // 128) * 128, 128)

    wvecs = [widx_v[pl.ds(g * L, L)] for g in range(BPW // L)]
    cvecs = [cidx_v[pl.ds(g * L, L)] for g in range(BPW // L)]

    def fire(i):
        buf = i % NBUF
        sem = sems[buf]
        wc = wvecs[i // L][i % L]
        cc = cvecs[i // L][i % L]
        pltpu.async_copy(tT_hbm.at[:, pl.ds(slab_start(wc), 128)],
                         wslab_v.at[buf], sem)
        pltpu.async_copy(tT_hbm.at[:, pl.ds(slab_start(cc), 128)],
                         cslab_v.at[buf], sem)

    for _i in range(NBUF):
        fire(_i)

    for g in range(BPW // L):
        res = jnp.zeros((L,), jnp.float32)
        for r16 in range(L):
            i = g * L + r16
            buf = i % NBUF
            pltpu.make_async_copy(
                tT_hbm.at[:, pl.ds(0, 128)], wslab_v.at[buf], sems[buf]).wait()
            pltpu.make_async_copy(
                tT_hbm.at[:, pl.ds(0, 128)], cslab_v.at[buf], sems[buf]).wait()
            wj = wvecs[g][r16] % 128
            cj = cvecs[g][r16] % 128
            wjv = jnp.zeros((L,), jnp.int32) + wj
            cjv = jnp.zeros((L,), jnp.int32) + cj
            acc = jnp.zeros((L,), jnp.float32)
            for k in range(EMB // L):
                rows = lane + (k * L)
                wcol = plsc.load_gather(wslab_v.at[buf], [rows, wjv])
                ccol = plsc.load_gather(cslab_v.at[buf], [rows, cjv])
                acc = acc + wcol * ccol
            if i + NBUF < BPW:
                fire(i + NBUF)
            for p in perms:
                acc = acc + _lane_shuffle(acc, p)
            res = jnp.where(lane == r16, acc, res)
        out_v[pl.ds(g * L, L)] = 1.0 / (1.0 + jnp.exp(-res))

    pltpu.sync_copy(out_v, out_hbm.at[pl.ds(base, BPW)])


NTBUF = 8
_ROWS_TC = B_TC // 128


def _w2v_tc_body(widx_s, cidx_s, tT_ref, out_ref, slabs_v, sems):
    lane = lax.broadcasted_iota(jnp.int32, (EMB, 128), 1)

    def fire(p):
        buf = lax.rem(p, NTBUF)
        wtc = (widx_s[p] // 128) * 128
        ctc = (cidx_s[p] // 128) * 128
        pltpu.make_async_copy(
            tT_ref.at[:, pl.ds(wtc, 128)], slabs_v.at[buf, 0],
            sems.at[buf]).start()
        pltpu.make_async_copy(
            tT_ref.at[:, pl.ds(ctc, 128)], slabs_v.at[buf, 1],
            sems.at[buf]).start()

    def prime(p, c):
        fire(p)
        return c

    lax.fori_loop(0, NTBUF, prime, 0)

    row_lane = lax.broadcasted_iota(jnp.int32, (1, 128), 1)

    def pair(p, row):
        buf = lax.rem(p, NTBUF)
        pltpu.make_async_copy(
            tT_ref.at[:, pl.ds(0, 128)], slabs_v.at[buf, 0],
            sems.at[buf]).wait()
        pltpu.make_async_copy(
            tT_ref.at[:, pl.ds(0, 128)], slabs_v.at[buf, 1],
            sems.at[buf]).wait()
        wj = widx_s[p] % 128
        cj = cidx_s[p] % 128
        wblk = slabs_v[buf, 0]
        cblk = slabs_v[buf, 1]
        wcol = jnp.sum(jnp.where(lane == wj, wblk, 0.0), axis=1)
        ccol = jnp.sum(jnp.where(lane == cj, cblk, 0.0), axis=1)
        dot = jnp.sum(wcol * ccol)

        @pl.when(p + NTBUF < B_TC)
        def _():
            fire(p + NTBUF)

        row = jnp.where(row_lane == lax.rem(p, 128),
                        1.0 / (1.0 + jnp.exp(-dot)), row)

        @pl.when(lax.rem(p, 128) == 127)
        def _():
            out_ref[pl.ds(p // 128, 1), :] = row

        return jnp.where(lax.rem(p, 128) == 127,
                         jnp.zeros((1, 128), jnp.float32), row)

    lax.fori_loop(0, B_TC, pair, jnp.zeros((1, 128), jnp.float32))


def _w2v_tc(tT, widx, cidx):
    return pl.pallas_call(
        _w2v_tc_body,
        grid_spec=pltpu.PrefetchScalarGridSpec(
            num_scalar_prefetch=2,
            grid=(1,),
            in_specs=[pl.BlockSpec(memory_space=pltpu.HBM)],
            out_specs=pl.BlockSpec(memory_space=pltpu.VMEM),
            scratch_shapes=[
                pltpu.VMEM((NTBUF, 2, EMB, 128), jnp.float32),
                pltpu.SemaphoreType.DMA((NTBUF,)),
            ],
        ),
        out_shape=jax.ShapeDtypeStruct((_ROWS_TC, 128), jnp.float32),
    )(widx, cidx, tT)


def kernel(word_vector, context_vector, table):
    widx = word_vector.reshape(B).astype(jnp.int32)
    cidx = context_vector.reshape(B).astype(jnp.int32)
    tT = table.T
    out_sc = _w2v_sc(tT, widx[B_TC:], cidx[B_TC:])
    out_tc = _w2v_tc(tT, widx[:B_TC], cidx[:B_TC]).reshape(B_TC)
    return jnp.concatenate([out_tc, out_sc])


# speedup vs baseline: 5.2908x; 5.2908x over previous
import functools

import jax
import jax.numpy as jnp
from jax import lax
from jax.experimental import pallas as pl
from jax.experimental.pallas import tpu as pltpu
from jax.experimental.pallas import tpu_sc as plsc

VOCAB = 1000000
EMB = 64
B = 4096
NC = 2
NS = 16
L = 16
NW = NC * NS
BPW = B // NW
NBUF = 7

_mesh = plsc.VectorSubcoreMesh(core_axis_name="c", subcore_axis_name="s")

_GATHER_DNUMS = lax.GatherDimensionNumbers(
    offset_dims=(), collapsed_slice_dims=(0,), start_index_map=(0,))


def _lane_shuffle(x, idx):
    return lax.gather(
        x, idx[:, None], _GATHER_DNUMS, slice_sizes=(1,),
        mode=lax.GatherScatterMode.PROMISE_IN_BOUNDS)


@functools.partial(
    pl.kernel,
    mesh=_mesh,
    compiler_params=pltpu.CompilerParams(needs_layout_passes=False),
    out_type=jax.ShapeDtypeStruct((B,), jnp.float32),
    scratch_types=[
        pltpu.VMEM((BPW,), jnp.int32),
        pltpu.VMEM((BPW,), jnp.int32),
        pltpu.VMEM((NBUF, EMB, 128), jnp.float32),
        pltpu.VMEM((NBUF, EMB, 128), jnp.float32),
        pltpu.VMEM((BPW,), jnp.float32),
        pltpu.SemaphoreType.DMA,
        pltpu.SemaphoreType.DMA,
        pltpu.SemaphoreType.DMA,
        pltpu.SemaphoreType.DMA,
        pltpu.SemaphoreType.DMA,
        pltpu.SemaphoreType.DMA,
        pltpu.SemaphoreType.DMA,
    ],
)
def _w2v_sc(tT_hbm, widx_hbm, cidx_hbm, out_hbm,
            widx_v, cidx_v, wslab_v, cslab_v, out_v, sem0, sem1, sem2, sem3, sem4, sem5, sem6):
    wid = lax.axis_index("s") * NC + lax.axis_index("c")
    base = wid * BPW
    pltpu.sync_copy(widx_hbm.at[pl.ds(base, BPW)], widx_v)
    pltpu.sync_copy(cidx_hbm.at[pl.ds(base, BPW)], cidx_v)

    sems = (sem0, sem1, sem2, sem3, sem4, sem5, sem6)
    lane = lax.iota(jnp.int32, L)
    perms = [jnp.bitwise_xor(lane, s) for s in (8, 4, 2, 1)]

    def slab_start(c):
        return pl.multiple_of((c // 128) * 128, 128)

    wvecs = [widx_v[pl.ds(g * L, L)] for g in range(BPW // L)]
    cvecs = [cidx_v[pl.ds(g * L, L)] for g in range(BPW // L)]

    def fire(i):
        buf = i % NBUF
        sem = sems[buf]
        wc = wvecs[i // L][i % L]
        cc = cvecs[i // L][i % L]
        pltpu.async_copy(tT_hbm.at[:, pl.ds(slab_start(wc), 128)],
                         wslab_v.at[buf], sem)
        pltpu.async_copy(tT_hbm.at[:, pl.ds(slab_start(cc), 128)],
                         cslab_v.at[buf], sem)

    for _i in range(NBUF):
        fire(_i)

    for g in range(BPW // L):
        res = jnp.zeros((L,), jnp.float32)
        for r16 in range(L):
            i = g * L + r16
            buf = i % NBUF
            pltpu.make_async_copy(
                tT_hbm.at[:, pl.ds(0, 128)], wslab_v.at[buf], sems[buf]).wait()
            pltpu.make_async_copy(
                tT_hbm.at[:, pl.ds(0, 128)], cslab_v.at[buf], sems[buf]).wait()
            wj = wvecs[g][r16] % 128
            cj = cvecs[g][r16] % 128
            wjv = jnp.zeros((L,), jnp.int32) + wj
            cjv = jnp.zeros((L,), jnp.int32) + cj
            acc = jnp.zeros((L,), jnp.float32)
            for k in range(EMB // L):
                rows = lane + (k * L)
                wcol = plsc.load_gather(wslab_v.at[buf], [rows, wjv])
                ccol = plsc.load_gather(cslab_v.at[buf], [rows, cjv])
                acc = acc + wcol * ccol
            if i + NBUF < BPW:
                fire(i + NBUF)
            for p in perms:
                acc = acc + _lane_shuffle(acc, p)
            res = jnp.where(lane == r16, acc, res)
        out_v[pl.ds(g * L, L)] = 1.0 / (1.0 + jnp.exp(-res))

    pltpu.sync_copy(out_v, out_hbm.at[pl.ds(base, BPW)])


def kernel(word_vector, context_vector, table):
    widx = word_vector.reshape(B).astype(jnp.int32)
    cidx = context_vector.reshape(B).astype(jnp.int32)
    return _w2v_sc(table.T, widx, cidx)
